# baseline (speedup 1.0000x reference)
import functools

import jax
import jax.numpy as jnp
from jax.experimental import pallas as pl
from jax.experimental.pallas import tpu as pltpu
from jax.experimental.pallas import tpu_sc as plsc

N = 10000
E = 320000
DIM = 128
HDIM = 64
LANES = 16

NC = 2
NS = 16
NW = NC * NS
EPW = E // NW
C = 80
KCH = EPW // C
ZR = 80
NZCH = N // ZR

_mesh = plsc.VectorSubcoreMesh(
    core_axis_name="c", subcore_axis_name="s", num_cores=NC, num_subcores=NS
)


@functools.partial(
    pl.kernel,
    out_type=jax.ShapeDtypeStruct((NC, N, HDIM), jnp.float32),
    mesh=_mesh,
    scratch_types=[
        pltpu.VMEM((KCH, C), jnp.int32),
        pltpu.VMEM((KCH, C), jnp.int32),
        pltpu.VMEM((KCH, C), jnp.float32),
        pltpu.VMEM((C, HDIM), jnp.float32),
        pltpu.VMEM((C, HDIM), jnp.float32),
        pltpu.VMEM((C, HDIM), jnp.float32),
        pltpu.VMEM((C, HDIM), jnp.float32),
        pltpu.VMEM((ZR, HDIM), jnp.float32),
        pltpu.VMEM_SHARED((N, HDIM), jnp.float32),
        pltpu.SemaphoreType.DMA,
        pltpu.SemaphoreType.DMA,
        pltpu.SemaphoreType.DMA,
        pltpu.SemaphoreType.DMA,
        pltpu.SemaphoreType.DMA,
        pltpu.SemaphoreType.DMA,
        pltpu.SemaphoreType.DMA,
        pltpu.SemaphoreType.DMA,
    ],
    compiler_params=pltpu.CompilerParams(use_tc_tiling_on_sc=False),
)
def _spmm_sc(table, gidx, sidx, vals, out,
             gidx_v, sidx_v, vals_v, buf0, buf1, buf2, buf3, zbuf, acc,
             gsem0, gsem1, gsem2, gsem3, ssem0, ssem1, ssem2, ssem3):
    core = jax.lax.axis_index("c")
    sub = jax.lax.axis_index("s")
    wid = core * NS + sub

    @pl.loop(0, ZR)
    def _zrow(r):
        for d in range(HDIM // LANES):
            zbuf[r, pl.ds(d * LANES, LANES)] = jnp.zeros((LANES,), jnp.float32)

    @pl.loop(0, (NZCH + NS - 1) // NS)
    def _zacc(j):
        cid = sub + j * NS

        @pl.when(cid < NZCH)
        def _():
            pltpu.sync_copy(zbuf, acc.at[pl.ds(cid * ZR, ZR)])

    pltpu.sync_copy(gidx.at[wid], gidx_v)
    pltpu.sync_copy(sidx.at[wid], sidx_v)
    pltpu.sync_copy(vals.at[wid], vals_v)

    plsc.subcore_barrier()

    bufs = (buf0, buf1, buf2, buf3)
    gsems = (gsem0, gsem1, gsem2, gsem3)
    ssems = (ssem0, ssem1, ssem2, ssem3)
    NBUF = 4

    def start_gather(k, b):
        pltpu.async_copy(table.at[gidx_v.at[k]], bufs[b], gsems[b])

    def wait_gather(k, b):
        pltpu.make_async_copy(table.at[gidx_v.at[k]], bufs[b], gsems[b]).wait()

    def start_scatter(k, b):
        pltpu.async_copy(bufs[b], acc.at[sidx_v.at[k]], ssems[b], add=True)

    def wait_scatter(k, b):
        pltpu.make_async_copy(bufs[b], acc.at[sidx_v.at[k]], ssems[b]).wait()

    def scale_chunk(kc, b):
        buf = bufs[b]

        @pl.loop(0, C // LANES)
        def _grp(g):
            vals16 = vals_v[kc, pl.ds(g * LANES, LANES)]
            e0 = g * LANES

            @plsc.parallel_loop(0, LANES, 1, unroll=4)
            def _edge(e2):
                v16 = jnp.take_along_axis(
                    vals16, jnp.full((LANES,), e2, jnp.int32), axis=0,
                    mode="promise_in_bounds")
                e = e0 + e2
                for d in range(HDIM // LANES):
                    sl = pl.ds(d * LANES, LANES)
                    buf[e, sl] = buf[e, sl] * v16

    def step(kc, b):
        nxt = kc + 1
        nb = (b + 1) % NBUF

        @pl.when(nxt < KCH)
        def _():
            @pl.when(nxt >= NBUF)
            def _():
                wait_scatter(nxt - NBUF, nb)

            start_gather(nxt, nb)

        wait_gather(kc, b)
        scale_chunk(kc, b)
        start_scatter(kc, b)

    start_gather(0, 0)

    @pl.loop(0, KCH - 1, step=NBUF)
    def _chunks(k):
        for i in range(NBUF):
            step(k + i, i)

    step(KCH - 1, (KCH - 1) % NBUF)
    for c in range(KCH - NBUF, KCH):
        wait_scatter(c, c % NBUF)

    plsc.subcore_barrier()

    @pl.loop(0, (NZCH + NS - 1) // NS)
    def _dump(j):
        cid = sub + j * NS

        @pl.when(cid < NZCH)
        def _():
            base = cid * ZR
            pltpu.sync_copy(acc.at[pl.ds(base, ZR)],
                            out.at[core, pl.ds(base, ZR)])


_BLK = 1000


def _mix_body(plo_ref, phi_ref, u_ref, w_ref, b_ref, olo_ref, ohi_ref):
    nm = jnp.concatenate(
        [plo_ref[0] + plo_ref[1], phi_ref[0] + phi_ref[1]], axis=1)
    w1 = w_ref[0:DIM, :]
    w2 = w_ref[DIM:2 * DIM, :]
    m = (jnp.dot(nm, w1, preferred_element_type=jnp.float32)
         + jnp.dot(nm * u_ref[...], w2, preferred_element_type=jnp.float32)
         + b_ref[...])
    olo_ref[...] = m[:, :HDIM]
    ohi_ref[...] = m[:, HDIM:]


def _sum2_body(qlo_ref, qhi_ref, o_ref):
    o_ref[:, 0:HDIM] = qlo_ref[0] + qlo_ref[1]
    o_ref[:, HDIM:DIM] = qhi_ref[0] + qhi_ref[1]


def _mix_tc(p_lo, p_hi, user_emb, W, b2):
    grid = N // _BLK
    return pl.pallas_call(
        _mix_body,
        out_shape=(jax.ShapeDtypeStruct((N, HDIM), jnp.float32),
                   jax.ShapeDtypeStruct((N, HDIM), jnp.float32)),
        grid=(grid,),
        in_specs=[
            pl.BlockSpec((NC, _BLK, HDIM), lambda i: (0, i, 0)),
            pl.BlockSpec((NC, _BLK, HDIM), lambda i: (0, i, 0)),
            pl.BlockSpec((_BLK, DIM), lambda i: (i, 0)),
            pl.BlockSpec((2 * DIM, DIM), lambda i: (0, 0)),
            pl.BlockSpec((1, DIM), lambda i: (0, 0)),
        ],
        out_specs=(pl.BlockSpec((_BLK, HDIM), lambda i: (i, 0)),
                   pl.BlockSpec((_BLK, HDIM), lambda i: (i, 0))),
    )(p_lo, p_hi, user_emb, W, b2)


def _sum2_tc(q_lo, q_hi):
    grid = N // _BLK
    return pl.pallas_call(
        _sum2_body,
        out_shape=jax.ShapeDtypeStruct((N, DIM), jnp.float32),
        grid=(grid,),
        in_specs=[
            pl.BlockSpec((NC, _BLK, HDIM), lambda i: (0, i, 0)),
            pl.BlockSpec((NC, _BLK, HDIM), lambda i: (0, i, 0)),
        ],
        out_specs=pl.BlockSpec((_BLK, DIM), lambda i: (i, 0)),
    )(q_lo, q_hi)


def kernel(user_emb, item_emb, hg_rows, hg_cols, hg_vals, W, b):
    rows3 = hg_rows.reshape(NW, KCH, C)
    cols3 = hg_cols.reshape(NW, KCH, C)
    vals3 = hg_vals.reshape(NW, KCH, C)

    item_lo = item_emb[:, :HDIM]
    item_hi = item_emb[:, HDIM:]

    p_lo = _spmm_sc(item_lo, cols3, rows3, vals3)
    p_hi = _spmm_sc(item_hi, cols3, rows3, vals3)
    msg_lo, msg_hi = _mix_tc(p_lo, p_hi, user_emb, W, b.reshape(1, DIM))
    q_lo = _spmm_sc(msg_lo, rows3, cols3, vals3)
    q_hi = _spmm_sc(msg_hi, rows3, cols3, vals3)
    norm_emb = _sum2_tc(q_lo, q_hi)
    msg = jnp.concatenate([msg_lo, msg_hi], axis=1)
    return (norm_emb, msg)


# speedup vs baseline: 1.0016x; 1.0016x over previous
import functools

import jax
import jax.numpy as jnp
from jax.experimental import pallas as pl
from jax.experimental.pallas import tpu as pltpu
from jax.experimental.pallas import tpu_sc as plsc

N = 10000
E = 320000
DIM = 128
HDIM = 64
LANES = 16

NC = 2
NS = 16
NW = NC * NS
EPW = E // NW
C = 80
KCH = EPW // C
ZR = 80
NZCH = N // ZR

_mesh = plsc.VectorSubcoreMesh(
    core_axis_name="c", subcore_axis_name="s", num_cores=NC, num_subcores=NS
)


@functools.partial(
    pl.kernel,
    out_type=jax.ShapeDtypeStruct((NC, N, HDIM), jnp.float32),
    mesh=_mesh,
    scratch_types=[
        pltpu.VMEM((KCH, C), jnp.int32),
        pltpu.VMEM((KCH, C), jnp.int32),
        pltpu.VMEM((KCH, C), jnp.float32),
        pltpu.VMEM((C, HDIM), jnp.float32),
        pltpu.VMEM((C, HDIM), jnp.float32),
        pltpu.VMEM((C, HDIM), jnp.float32),
        pltpu.VMEM((C, HDIM), jnp.float32),
        pltpu.VMEM((ZR, HDIM), jnp.float32),
        pltpu.VMEM_SHARED((N, HDIM), jnp.float32),
        pltpu.SemaphoreType.DMA,
        pltpu.SemaphoreType.DMA,
        pltpu.SemaphoreType.DMA,
        pltpu.SemaphoreType.DMA,
        pltpu.SemaphoreType.DMA,
        pltpu.SemaphoreType.DMA,
        pltpu.SemaphoreType.DMA,
        pltpu.SemaphoreType.DMA,
    ],
    compiler_params=pltpu.CompilerParams(use_tc_tiling_on_sc=False),
)
def _spmm_sc(table, gidx, sidx, vals, out,
             gidx_v, sidx_v, vals_v, buf0, buf1, buf2, buf3, zbuf, acc,
             gsem0, gsem1, gsem2, gsem3, ssem0, ssem1, ssem2, ssem3):
    core = jax.lax.axis_index("c")
    sub = jax.lax.axis_index("s")
    wid = core * NS + sub

    @pl.loop(0, ZR)
    def _zrow(r):
        for d in range(HDIM // LANES):
            zbuf[r, pl.ds(d * LANES, LANES)] = jnp.zeros((LANES,), jnp.float32)

    @pl.loop(0, (NZCH + NS - 1) // NS)
    def _zacc(j):
        cid = sub + j * NS

        @pl.when(cid < NZCH)
        def _():
            pltpu.sync_copy(zbuf, acc.at[pl.ds(cid * ZR, ZR)])

    pltpu.sync_copy(gidx.at[wid], gidx_v)
    pltpu.sync_copy(sidx.at[wid], sidx_v)
    pltpu.sync_copy(vals.at[wid], vals_v)

    plsc.subcore_barrier()

    bufs = (buf0, buf1, buf2, buf3)
    gsems = (gsem0, gsem1, gsem2, gsem3)
    ssems = (ssem0, ssem1, ssem2, ssem3)
    NBUF = 4

    def start_gather(k, b):
        pltpu.async_copy(table.at[gidx_v.at[k]], bufs[b], gsems[b])

    def wait_gather(k, b):
        pltpu.make_async_copy(table.at[gidx_v.at[k]], bufs[b], gsems[b]).wait()

    def start_scatter(k, b):
        pltpu.async_copy(bufs[b], acc.at[sidx_v.at[k]], ssems[b], add=True)

    def wait_scatter(k, b):
        pltpu.make_async_copy(bufs[b], acc.at[sidx_v.at[k]], ssems[b]).wait()

    def scale_chunk(kc, b):
        buf = bufs[b]

        @pl.loop(0, C // LANES)
        def _grp(g):
            vals16 = vals_v[kc, pl.ds(g * LANES, LANES)]
            e0 = g * LANES

            @pl.loop(0, LANES, unroll=4)
            def _edge(e2):
                v16 = jnp.take_along_axis(
                    vals16, jnp.full((LANES,), e2, jnp.int32), axis=0,
                    mode="promise_in_bounds")
                e = e0 + e2
                for d in range(HDIM // LANES):
                    sl = pl.ds(d * LANES, LANES)
                    buf[e, sl] = buf[e, sl] * v16

    def step(kc, b):
        nxt = kc + 1
        nb = (b + 1) % NBUF

        @pl.when(nxt < KCH)
        def _():
            @pl.when(nxt >= NBUF)
            def _():
                wait_scatter(nxt - NBUF, nb)

            start_gather(nxt, nb)

        wait_gather(kc, b)
        scale_chunk(kc, b)
        start_scatter(kc, b)

    start_gather(0, 0)

    @pl.loop(0, KCH - 1, step=NBUF)
    def _chunks(k):
        for i in range(NBUF):
            step(k + i, i)

    step(KCH - 1, (KCH - 1) % NBUF)
    for c in range(KCH - NBUF, KCH):
        wait_scatter(c, c % NBUF)

    plsc.subcore_barrier()

    @pl.loop(0, (NZCH + NS - 1) // NS)
    def _dump(j):
        cid = sub + j * NS

        @pl.when(cid < NZCH)
        def _():
            base = cid * ZR
            pltpu.sync_copy(acc.at[pl.ds(base, ZR)],
                            out.at[core, pl.ds(base, ZR)])


_BLK = 1000


def _mix_body(plo_ref, phi_ref, u_ref, w_ref, b_ref, olo_ref, ohi_ref):
    nm = jnp.concatenate(
        [plo_ref[0] + plo_ref[1], phi_ref[0] + phi_ref[1]], axis=1)
    w1 = w_ref[0:DIM, :]
    w2 = w_ref[DIM:2 * DIM, :]
    m = (jnp.dot(nm, w1, preferred_element_type=jnp.float32)
         + jnp.dot(nm * u_ref[...], w2, preferred_element_type=jnp.float32)
         + b_ref[...])
    olo_ref[...] = m[:, :HDIM]
    ohi_ref[...] = m[:, HDIM:]


def _sum2_body(qlo_ref, qhi_ref, o_ref):
    o_ref[:, 0:HDIM] = qlo_ref[0] + qlo_ref[1]
    o_ref[:, HDIM:DIM] = qhi_ref[0] + qhi_ref[1]


def _mix_tc(p_lo, p_hi, user_emb, W, b2):
    grid = N // _BLK
    return pl.pallas_call(
        _mix_body,
        out_shape=(jax.ShapeDtypeStruct((N, HDIM), jnp.float32),
                   jax.ShapeDtypeStruct((N, HDIM), jnp.float32)),
        grid=(grid,),
        in_specs=[
            pl.BlockSpec((NC, _BLK, HDIM), lambda i: (0, i, 0)),
            pl.BlockSpec((NC, _BLK, HDIM), lambda i: (0, i, 0)),
            pl.BlockSpec((_BLK, DIM), lambda i: (i, 0)),
            pl.BlockSpec((2 * DIM, DIM), lambda i: (0, 0)),
            pl.BlockSpec((1, DIM), lambda i: (0, 0)),
        ],
        out_specs=(pl.BlockSpec((_BLK, HDIM), lambda i: (i, 0)),
                   pl.BlockSpec((_BLK, HDIM), lambda i: (i, 0))),
    )(p_lo, p_hi, user_emb, W, b2)


def _sum2_tc(q_lo, q_hi):
    grid = N // _BLK
    return pl.pallas_call(
        _sum2_body,
        out_shape=jax.ShapeDtypeStruct((N, DIM), jnp.float32),
        grid=(grid,),
        in_specs=[
            pl.BlockSpec((NC, _BLK, HDIM), lambda i: (0, i, 0)),
            pl.BlockSpec((NC, _BLK, HDIM), lambda i: (0, i, 0)),
        ],
        out_specs=pl.BlockSpec((_BLK, DIM), lambda i: (i, 0)),
    )(q_lo, q_hi)


def kernel(user_emb, item_emb, hg_rows, hg_cols, hg_vals, W, b):
    rows3 = hg_rows.reshape(NW, KCH, C)
    cols3 = hg_cols.reshape(NW, KCH, C)
    vals3 = hg_vals.reshape(NW, KCH, C)

    item_lo = item_emb[:, :HDIM]
    item_hi = item_emb[:, HDIM:]

    p_lo = _spmm_sc(item_lo, cols3, rows3, vals3)
    p_hi = _spmm_sc(item_hi, cols3, rows3, vals3)
    msg_lo, msg_hi = _mix_tc(p_lo, p_hi, user_emb, W, b.reshape(1, DIM))
    q_lo = _spmm_sc(msg_lo, rows3, cols3, vals3)
    q_hi = _spmm_sc(msg_hi, rows3, cols3, vals3)
    norm_emb = _sum2_tc(q_lo, q_hi)
    msg = jnp.concatenate([msg_lo, msg_hi], axis=1)
    return (norm_emb, msg)


# speedup vs baseline: 1.0658x; 1.0641x over previous
import functools

import jax
import jax.numpy as jnp
from jax.experimental import pallas as pl
from jax.experimental.pallas import tpu as pltpu
from jax.experimental.pallas import tpu_sc as plsc

N = 10000
E = 320000
DIM = 128
HDIM = 64
LANES = 16

NC = 2
NS = 16
EPW = E // NS
C = 80
KCH = EPW // C
NBUF = 4
ZR = 80
NZCH = N // ZR

_mesh = plsc.VectorSubcoreMesh(
    core_axis_name="c", subcore_axis_name="s", num_cores=NC, num_subcores=NS
)


@functools.partial(
    pl.kernel,
    out_type=jax.ShapeDtypeStruct((NC, N, HDIM), jnp.float32),
    mesh=_mesh,
    scratch_types=[
        pltpu.VMEM((KCH, C), jnp.int32),
        pltpu.VMEM((KCH, C), jnp.int32),
        pltpu.VMEM((KCH, C), jnp.float32),
        pltpu.VMEM((C, HDIM), jnp.float32),
        pltpu.VMEM((C, HDIM), jnp.float32),
        pltpu.VMEM((C, HDIM), jnp.float32),
        pltpu.VMEM((C, HDIM), jnp.float32),
        pltpu.VMEM((ZR, HDIM), jnp.float32),
        pltpu.VMEM_SHARED((N, HDIM), jnp.float32),
        pltpu.SemaphoreType.DMA,
        pltpu.SemaphoreType.DMA,
        pltpu.SemaphoreType.DMA,
        pltpu.SemaphoreType.DMA,
        pltpu.SemaphoreType.DMA,
        pltpu.SemaphoreType.DMA,
        pltpu.SemaphoreType.DMA,
        pltpu.SemaphoreType.DMA,
    ],
    compiler_params=pltpu.CompilerParams(use_tc_tiling_on_sc=False),
)
def _spmm_sc(tables, gidx, sidx, vals, out,
             gidx_v, sidx_v, vals_v, buf0, buf1, buf2, buf3, zbuf, acc,
             gsem0, gsem1, gsem2, gsem3, ssem0, ssem1, ssem2, ssem3):
    core = jax.lax.axis_index("c")
    sub = jax.lax.axis_index("s")
    table = tables.at[core]

    @pl.loop(0, ZR)
    def _zrow(r):
        for d in range(HDIM // LANES):
            zbuf[r, pl.ds(d * LANES, LANES)] = jnp.zeros((LANES,), jnp.float32)

    @pl.loop(0, (NZCH + NS - 1) // NS)
    def _zacc(j):
        cid = sub + j * NS

        @pl.when(cid < NZCH)
        def _():
            pltpu.sync_copy(zbuf, acc.at[pl.ds(cid * ZR, ZR)])

    pltpu.sync_copy(gidx.at[sub], gidx_v)
    pltpu.sync_copy(sidx.at[sub], sidx_v)
    pltpu.sync_copy(vals.at[sub], vals_v)

    plsc.subcore_barrier()

    bufs = (buf0, buf1, buf2, buf3)
    gsems = (gsem0, gsem1, gsem2, gsem3)
    ssems = (ssem0, ssem1, ssem2, ssem3)

    def start_gather(k, b):
        pltpu.async_copy(table.at[gidx_v.at[k]], bufs[b], gsems[b])

    def wait_gather(k, b):
        pltpu.make_async_copy(table.at[gidx_v.at[k]], bufs[b], gsems[b]).wait()

    def start_scatter(k, b):
        pltpu.async_copy(bufs[b], acc.at[sidx_v.at[k]], ssems[b], add=True)

    def wait_scatter(k, b):
        pltpu.make_async_copy(bufs[b], acc.at[sidx_v.at[k]], ssems[b]).wait()

    def scale_chunk(kc, b):
        buf = bufs[b]

        @pl.loop(0, C // LANES)
        def _grp(g):
            vals16 = vals_v[kc, pl.ds(g * LANES, LANES)]
            e0 = g * LANES

            @pl.loop(0, LANES, unroll=4)
            def _edge(e2):
                v16 = jnp.take_along_axis(
                    vals16, jnp.full((LANES,), e2, jnp.int32), axis=0,
                    mode="promise_in_bounds")
                e = e0 + e2
                for d in range(HDIM // LANES):
                    sl = pl.ds(d * LANES, LANES)
                    buf[e, sl] = buf[e, sl] * v16

    def step(kc, b):
        nxt = kc + 1
        nb = (b + 1) % NBUF

        @pl.when(nxt < KCH)
        def _():
            @pl.when(nxt >= NBUF)
            def _():
                wait_scatter(nxt - NBUF, nb)

            start_gather(nxt, nb)

        wait_gather(kc, b)
        scale_chunk(kc, b)
        start_scatter(kc, b)

    start_gather(0, 0)
    KMAIN = (KCH - 2) // NBUF * NBUF

    @pl.loop(0, KMAIN, step=NBUF)
    def _chunks(k):
        for i in range(NBUF):
            step(k + i, i)

    for kc in range(KMAIN, KCH):
        step(kc, kc % NBUF)
    for c in range(KCH - NBUF, KCH):
        wait_scatter(c, c % NBUF)

    plsc.subcore_barrier()

    @pl.loop(0, (NZCH + NS - 1) // NS)
    def _dump(j):
        cid = sub + j * NS

        @pl.when(cid < NZCH)
        def _():
            base = cid * ZR
            pltpu.sync_copy(acc.at[pl.ds(base, ZR)],
                            out.at[core, pl.ds(base, ZR)])


_BLK = 1000


def _mix_body(nm_ref, u_ref, w_ref, b_ref, o_ref):
    nm = jnp.concatenate([nm_ref[0], nm_ref[1]], axis=1)
    w1 = w_ref[0:DIM, :]
    w2 = w_ref[DIM:2 * DIM, :]
    m = (jnp.dot(nm, w1, preferred_element_type=jnp.float32)
         + jnp.dot(nm * u_ref[...], w2, preferred_element_type=jnp.float32)
         + b_ref[...])
    o_ref[0] = m[:, :HDIM]
    o_ref[1] = m[:, HDIM:]


def _fin_body(q_ref, mh_ref, on_ref, om_ref):
    on_ref[:, 0:HDIM] = q_ref[0]
    on_ref[:, HDIM:DIM] = q_ref[1]
    om_ref[:, 0:HDIM] = mh_ref[0]
    om_ref[:, HDIM:DIM] = mh_ref[1]


def _mix_tc(nm, user_emb, W, b2):
    grid = N // _BLK
    return pl.pallas_call(
        _mix_body,
        out_shape=jax.ShapeDtypeStruct((NC, N, HDIM), jnp.float32),
        grid=(grid,),
        in_specs=[
            pl.BlockSpec((NC, _BLK, HDIM), lambda i: (0, i, 0)),
            pl.BlockSpec((_BLK, DIM), lambda i: (i, 0)),
            pl.BlockSpec((2 * DIM, DIM), lambda i: (0, 0)),
            pl.BlockSpec((1, DIM), lambda i: (0, 0)),
        ],
        out_specs=pl.BlockSpec((NC, _BLK, HDIM), lambda i: (0, i, 0)),
    )(nm, user_emb, W, b2)


def _fin_tc(q, msgh):
    grid = N // _BLK
    return pl.pallas_call(
        _fin_body,
        out_shape=(jax.ShapeDtypeStruct((N, DIM), jnp.float32),
                   jax.ShapeDtypeStruct((N, DIM), jnp.float32)),
        grid=(grid,),
        in_specs=[
            pl.BlockSpec((NC, _BLK, HDIM), lambda i: (0, i, 0)),
            pl.BlockSpec((NC, _BLK, HDIM), lambda i: (0, i, 0)),
        ],
        out_specs=(pl.BlockSpec((_BLK, DIM), lambda i: (i, 0)),
                   pl.BlockSpec((_BLK, DIM), lambda i: (i, 0))),
    )(q, msgh)


def kernel(user_emb, item_emb, hg_rows, hg_cols, hg_vals, W, b):
    rows3 = hg_rows.reshape(NS, KCH, C)
    cols3 = hg_cols.reshape(NS, KCH, C)
    vals3 = hg_vals.reshape(NS, KCH, C)

    item_halves = jnp.stack([item_emb[:, :HDIM], item_emb[:, HDIM:]])

    nm = _spmm_sc(item_halves, cols3, rows3, vals3)
    msgh = _mix_tc(nm, user_emb, W, b.reshape(1, DIM))
    q = _spmm_sc(msgh, rows3, cols3, vals3)
    norm_emb, msg = _fin_tc(q, msgh)
    return (norm_emb, msg)


# speedup vs baseline: 1.1612x; 1.0895x over previous
import functools

import jax
import jax.numpy as jnp
from jax.experimental import pallas as pl
from jax.experimental.pallas import tpu as pltpu
from jax.experimental.pallas import tpu_sc as plsc

N = 10000
E = 320000
DIM = 128
HDIM = 64
LANES = 16

NC = 2
NS = 16
EPW = E // NS
C = 80
KCH = EPW // C
NBUF = 4
ZR = 80
NZCH = N // ZR

_mesh = plsc.VectorSubcoreMesh(
    core_axis_name="c", subcore_axis_name="s", num_cores=NC, num_subcores=NS
)


@functools.partial(
    pl.kernel,
    out_type=jax.ShapeDtypeStruct((N, DIM), jnp.float32),
    mesh=_mesh,
    scratch_types=[
        pltpu.VMEM((KCH, C), jnp.int32),
        pltpu.VMEM((KCH, C), jnp.int32),
        pltpu.VMEM((KCH, C), jnp.float32),
        pltpu.VMEM((C, HDIM), jnp.float32),
        pltpu.VMEM((C, HDIM), jnp.float32),
        pltpu.VMEM((C, HDIM), jnp.float32),
        pltpu.VMEM((C, HDIM), jnp.float32),
        pltpu.VMEM((ZR, HDIM), jnp.float32),
        pltpu.VMEM_SHARED((N, HDIM), jnp.float32),
        pltpu.SemaphoreType.DMA,
        pltpu.SemaphoreType.DMA,
        pltpu.SemaphoreType.DMA,
        pltpu.SemaphoreType.DMA,
        pltpu.SemaphoreType.DMA,
        pltpu.SemaphoreType.DMA,
        pltpu.SemaphoreType.DMA,
        pltpu.SemaphoreType.DMA,
    ],
    compiler_params=pltpu.CompilerParams(use_tc_tiling_on_sc=False),
)
def _spmm_sc(tables, gidx, sidx, vals, out,
             gidx_v, sidx_v, vals_v, buf0, buf1, buf2, buf3, zbuf, acc,
             gsem0, gsem1, gsem2, gsem3, ssem0, ssem1, ssem2, ssem3):
    core = jax.lax.axis_index("c")
    sub = jax.lax.axis_index("s")
    table = tables.at[core]

    @pl.loop(0, ZR)
    def _zrow(r):
        for d in range(HDIM // LANES):
            zbuf[r, pl.ds(d * LANES, LANES)] = jnp.zeros((LANES,), jnp.float32)

    @pl.loop(0, (NZCH + NS - 1) // NS)
    def _zacc(j):
        cid = sub + j * NS

        @pl.when(cid < NZCH)
        def _():
            pltpu.sync_copy(zbuf, acc.at[pl.ds(cid * ZR, ZR)])

    pltpu.sync_copy(gidx.at[sub], gidx_v)
    pltpu.sync_copy(sidx.at[sub], sidx_v)
    pltpu.sync_copy(vals.at[sub], vals_v)

    plsc.subcore_barrier()

    bufs = (buf0, buf1, buf2, buf3)
    gsems = (gsem0, gsem1, gsem2, gsem3)
    ssems = (ssem0, ssem1, ssem2, ssem3)

    def start_gather(k, b):
        pltpu.async_copy(table.at[gidx_v.at[k]], bufs[b], gsems[b])

    def wait_gather(k, b):
        pltpu.make_async_copy(table.at[gidx_v.at[k]], bufs[b], gsems[b]).wait()

    def start_scatter(k, b):
        pltpu.async_copy(bufs[b], acc.at[sidx_v.at[k]], ssems[b], add=True)

    def wait_scatter(k, b):
        pltpu.make_async_copy(bufs[b], acc.at[sidx_v.at[k]], ssems[b]).wait()

    def scale_chunk(kc, b):
        buf = bufs[b]

        @pl.loop(0, C // LANES)
        def _grp(g):
            vals16 = vals_v[kc, pl.ds(g * LANES, LANES)]
            e0 = g * LANES

            @pl.loop(0, LANES, unroll=4)
            def _edge(e2):
                v16 = jnp.take_along_axis(
                    vals16, jnp.full((LANES,), e2, jnp.int32), axis=0,
                    mode="promise_in_bounds")
                e = e0 + e2
                for d in range(HDIM // LANES):
                    sl = pl.ds(d * LANES, LANES)
                    buf[e, sl] = buf[e, sl] * v16

    def step(kc, b):
        nxt = kc + 1
        nb = (b + 1) % NBUF

        @pl.when(nxt < KCH)
        def _():
            @pl.when(nxt >= NBUF)
            def _():
                wait_scatter(nxt - NBUF, nb)

            start_gather(nxt, nb)

        wait_gather(kc, b)
        scale_chunk(kc, b)
        start_scatter(kc, b)

    start_gather(0, 0)
    KMAIN = (KCH - 2) // NBUF * NBUF

    @pl.loop(0, KMAIN, step=NBUF)
    def _chunks(k):
        for i in range(NBUF):
            step(k + i, i)

    for kc in range(KMAIN, KCH):
        step(kc, kc % NBUF)
    for c in range(KCH - NBUF, KCH):
        wait_scatter(c, c % NBUF)

    plsc.subcore_barrier()

    @pl.loop(0, (NZCH + NS - 1) // NS)
    def _dump(j):
        cid = sub + j * NS

        @pl.when(cid < NZCH)
        def _():
            base = cid * ZR
            pltpu.sync_copy(
                acc.at[pl.ds(base, ZR)],
                out.at[pl.ds(base, ZR), pl.ds(core * HDIM, HDIM)])


_BLK = 1000


def _mix_body(nm_ref, u_ref, w_ref, b_ref, o_ref, oh_ref):
    nm = nm_ref[...]
    w1 = w_ref[0:DIM, :]
    w2 = w_ref[DIM:2 * DIM, :]
    m = (jnp.dot(nm, w1, preferred_element_type=jnp.float32)
         + jnp.dot(nm * u_ref[...], w2, preferred_element_type=jnp.float32)
         + b_ref[...])
    o_ref[...] = m
    oh_ref[0] = m[:, :HDIM]
    oh_ref[1] = m[:, HDIM:]


def _mix_tc(nm, user_emb, W, b2):
    grid = N // _BLK
    return pl.pallas_call(
        _mix_body,
        out_shape=(jax.ShapeDtypeStruct((N, DIM), jnp.float32),
                   jax.ShapeDtypeStruct((NC, N, HDIM), jnp.float32)),
        grid=(grid,),
        in_specs=[
            pl.BlockSpec((_BLK, DIM), lambda i: (i, 0)),
            pl.BlockSpec((_BLK, DIM), lambda i: (i, 0)),
            pl.BlockSpec((2 * DIM, DIM), lambda i: (0, 0)),
            pl.BlockSpec((1, DIM), lambda i: (0, 0)),
        ],
        out_specs=(pl.BlockSpec((_BLK, DIM), lambda i: (i, 0)),
                   pl.BlockSpec((NC, _BLK, HDIM), lambda i: (0, i, 0))),
    )(nm, user_emb, W, b2)


def kernel(user_emb, item_emb, hg_rows, hg_cols, hg_vals, W, b):
    rows3 = hg_rows.reshape(NS, KCH, C)
    cols3 = hg_cols.reshape(NS, KCH, C)
    vals3 = hg_vals.reshape(NS, KCH, C)

    item_halves = jnp.stack([item_emb[:, :HDIM], item_emb[:, HDIM:]])

    nm = _spmm_sc(item_halves, cols3, rows3, vals3)
    msg, msgh = _mix_tc(nm, user_emb, W, b.reshape(1, DIM))
    norm_emb = _spmm_sc(msgh, rows3, cols3, vals3)
    return (norm_emb, msg)


# speedup vs baseline: 1.2056x; 1.0382x over previous
import functools

import jax
import jax.numpy as jnp
from jax.experimental import pallas as pl
from jax.experimental.pallas import tpu as pltpu
from jax.experimental.pallas import tpu_sc as plsc

N = 10000
E = 320000
DIM = 128
HDIM = 64
LANES = 16

NC = 2
NS = 16
EPW = E // NS
C = 80
KCH = EPW // C
NBUF = 4
ZR = 80
NZCH = N // ZR

_mesh = plsc.VectorSubcoreMesh(
    core_axis_name="c", subcore_axis_name="s", num_cores=NC, num_subcores=NS
)


def _make_spmm(interleaved):
    @functools.partial(
        pl.kernel,
        out_type=jax.ShapeDtypeStruct((N, DIM), jnp.float32),
        mesh=_mesh,
        scratch_types=[
            pltpu.VMEM((KCH, C), jnp.int32),
            pltpu.VMEM((KCH, C), jnp.int32),
            pltpu.VMEM((KCH, C), jnp.float32),
            pltpu.VMEM((C, HDIM), jnp.float32),
            pltpu.VMEM((C, HDIM), jnp.float32),
            pltpu.VMEM((C, HDIM), jnp.float32),
            pltpu.VMEM((C, HDIM), jnp.float32),
            pltpu.VMEM((ZR, HDIM), jnp.float32),
            pltpu.VMEM_SHARED((N, HDIM), jnp.float32),
            pltpu.SemaphoreType.DMA,
            pltpu.SemaphoreType.DMA,
            pltpu.SemaphoreType.DMA,
            pltpu.SemaphoreType.DMA,
            pltpu.SemaphoreType.DMA,
            pltpu.SemaphoreType.DMA,
            pltpu.SemaphoreType.DMA,
            pltpu.SemaphoreType.DMA,
        ],
        compiler_params=pltpu.CompilerParams(use_tc_tiling_on_sc=False),
    )
    def _spmm_sc(table, gidx, sidx, vals, out,
                 gidx_v, sidx_v, vals_v, buf0, buf1, buf2, buf3, zbuf, acc,
                 gsem0, gsem1, gsem2, gsem3, ssem0, ssem1, ssem2, ssem3):
        _spmm_body(interleaved, table, gidx, sidx, vals, out,
                   gidx_v, sidx_v, vals_v, buf0, buf1, buf2, buf3, zbuf, acc,
                   gsem0, gsem1, gsem2, gsem3, ssem0, ssem1, ssem2, ssem3)

    return _spmm_sc


def _spmm_body(interleaved, table, gidx, sidx, vals, out,
               gidx_v, sidx_v, vals_v, buf0, buf1, buf2, buf3, zbuf, acc,
               gsem0, gsem1, gsem2, gsem3, ssem0, ssem1, ssem2, ssem3):
    core = jax.lax.axis_index("c")
    sub = jax.lax.axis_index("s")

    @pl.loop(0, ZR)
    def _zrow(r):
        for d in range(HDIM // LANES):
            zbuf[r, pl.ds(d * LANES, LANES)] = jnp.zeros((LANES,), jnp.float32)

    @pl.loop(0, (NZCH + NS - 1) // NS)
    def _zacc(j):
        cid = sub + j * NS

        @pl.when(cid < NZCH)
        def _():
            pltpu.sync_copy(zbuf, acc.at[pl.ds(cid * ZR, ZR)])

    pltpu.sync_copy(gidx.at[sub], gidx_v)
    pltpu.sync_copy(sidx.at[sub], sidx_v)
    pltpu.sync_copy(vals.at[sub], vals_v)

    @pl.loop(0, KCH)
    def _tidx(k):
        for g in range(C // LANES):
            sl = pl.ds(g * LANES, LANES)
            if interleaved:
                gidx_v[k, sl] = gidx_v[k, sl] * 2 + core
            else:
                gidx_v[k, sl] = gidx_v[k, sl] + core * N

    plsc.subcore_barrier()

    bufs = (buf0, buf1, buf2, buf3)
    gsems = (gsem0, gsem1, gsem2, gsem3)
    ssems = (ssem0, ssem1, ssem2, ssem3)

    def start_gather(k, b):
        pltpu.async_copy(table.at[gidx_v.at[k]], bufs[b], gsems[b])

    def wait_gather(k, b):
        pltpu.make_async_copy(table.at[gidx_v.at[k]], bufs[b], gsems[b]).wait()

    def start_scatter(k, b):
        pltpu.async_copy(bufs[b], acc.at[sidx_v.at[k]], ssems[b], add=True)

    def wait_scatter(k, b):
        pltpu.make_async_copy(bufs[b], acc.at[sidx_v.at[k]], ssems[b]).wait()

    def scale_chunk(kc, b):
        buf = bufs[b]

        @pl.loop(0, C // LANES)
        def _grp(g):
            vals16 = vals_v[kc, pl.ds(g * LANES, LANES)]
            e0 = g * LANES

            @pl.loop(0, LANES, unroll=4)
            def _edge(e2):
                v16 = jnp.take_along_axis(
                    vals16, jnp.full((LANES,), e2, jnp.int32), axis=0,
                    mode="promise_in_bounds")
                e = e0 + e2
                for d in range(HDIM // LANES):
                    sl = pl.ds(d * LANES, LANES)
                    buf[e, sl] = buf[e, sl] * v16

    def step(kc, b):
        nxt = kc + 1
        nb = (b + 1) % NBUF

        @pl.when(nxt < KCH)
        def _():
            @pl.when(nxt >= NBUF)
            def _():
                wait_scatter(nxt - NBUF, nb)

            start_gather(nxt, nb)

        wait_gather(kc, b)
        scale_chunk(kc, b)
        start_scatter(kc, b)

    start_gather(0, 0)
    KMAIN = (KCH - 2) // NBUF * NBUF

    @pl.loop(0, KMAIN, step=NBUF)
    def _chunks(k):
        for i in range(NBUF):
            step(k + i, i)

    for kc in range(KMAIN, KCH):
        step(kc, kc % NBUF)
    for c in range(KCH - NBUF, KCH):
        wait_scatter(c, c % NBUF)

    plsc.subcore_barrier()

    @pl.loop(0, (NZCH + NS - 1) // NS)
    def _dump(j):
        cid = sub + j * NS

        @pl.when(cid < NZCH)
        def _():
            base = cid * ZR
            pltpu.sync_copy(
                acc.at[pl.ds(base, ZR)],
                out.at[pl.ds(base, ZR), pl.ds(core * HDIM, HDIM)])


_BLK = 1000


def _mix_body(nm_ref, u_ref, w_ref, b_ref, o_ref, oh_ref):
    nm = nm_ref[...]
    w1 = w_ref[0:DIM, :]
    w2 = w_ref[DIM:2 * DIM, :]
    m = (jnp.dot(nm, w1, preferred_element_type=jnp.float32)
         + jnp.dot(nm * u_ref[...], w2, preferred_element_type=jnp.float32)
         + b_ref[...])
    o_ref[...] = m
    oh_ref[0] = m[:, :HDIM]
    oh_ref[1] = m[:, HDIM:]


def _mix_tc(nm, user_emb, W, b2):
    grid = N // _BLK
    return pl.pallas_call(
        _mix_body,
        out_shape=(jax.ShapeDtypeStruct((N, DIM), jnp.float32),
                   jax.ShapeDtypeStruct((NC, N, HDIM), jnp.float32)),
        grid=(grid,),
        in_specs=[
            pl.BlockSpec((_BLK, DIM), lambda i: (i, 0)),
            pl.BlockSpec((_BLK, DIM), lambda i: (i, 0)),
            pl.BlockSpec((2 * DIM, DIM), lambda i: (0, 0)),
            pl.BlockSpec((1, DIM), lambda i: (0, 0)),
        ],
        out_specs=(pl.BlockSpec((_BLK, DIM), lambda i: (i, 0)),
                   pl.BlockSpec((NC, _BLK, HDIM), lambda i: (0, i, 0))),
    )(nm, user_emb, W, b2)


_spmm_interleaved = _make_spmm(True)
_spmm_stacked = _make_spmm(False)


def kernel(user_emb, item_emb, hg_rows, hg_cols, hg_vals, W, b):
    rows3 = hg_rows.reshape(NS, KCH, C)
    cols3 = hg_cols.reshape(NS, KCH, C)
    vals3 = hg_vals.reshape(NS, KCH, C)

    nm = _spmm_interleaved(item_emb.reshape(2 * N, HDIM), cols3, rows3, vals3)
    msg, msgh = _mix_tc(nm, user_emb, W, b.reshape(1, DIM))
    norm_emb = _spmm_stacked(msgh.reshape(2 * N, HDIM), rows3, cols3, vals3)
    return (norm_emb, msg)
